# baseline (speedup 1.0000x reference)
import functools

import jax
import jax.numpy as jnp
from jax import lax
from jax.experimental import pallas as pl
from jax.experimental.pallas import tpu as pltpu
from jax.experimental.pallas import tpu_sc as plsc

N_USERS = 25000
N_NODES = 50000
EMB_DIM = 64
N_EDGES = 800000
BATCH = 4096

NC = 2
NS = 16
HALF = N_NODES // NC
GROUPS = 6272
PAD_EDGES = GROUPS * 128 - N_EDGES
GP_TILE = GROUPS // NS
CHUNK_G = 8

N_CHUNKS = GP_TILE // CHUNK_G

PAIR_HALF = HALF // 2
ACC_P = 12544
ZSHARE_P = ACC_P // NS
TRASH_P = 12540

_mesh = plsc.VectorSubcoreMesh(core_axis_name="c", subcore_axis_name="s")


@functools.partial(
    pl.kernel,
    out_type=jax.ShapeDtypeStruct((NC, ACC_P, 128), jnp.float32),
    mesh=_mesh,
    scratch_types=[
        pltpu.VMEM((CHUNK_G, 128), jnp.int32),
        pltpu.VMEM((CHUNK_G, 128), jnp.int32),
        pltpu.VMEM((128, 128), jnp.float32),
        pltpu.VMEM_SHARED((ACC_P, 128), jnp.float32),
        pltpu.SemaphoreType.DMA,
    ],
)
def _sc_aggregate(gidx2d, lcol4, table, zerosp, agg_out,
                  gi_v, lq_v, rows_v, acc, sem):
    c = lax.axis_index("c")
    s = lax.axis_index("s")
    pltpu.sync_copy(zerosp, acc.at[pl.ds(s * ZSHARE_P, ZSHARE_P)])
    plsc.subcore_barrier()

    def process(g0):
        pltpu.sync_copy(gidx2d.at[pl.ds(g0, CHUNK_G)], gi_v)
        pltpu.sync_copy(lcol4.at[c, pl.ds(g0, CHUNK_G)], lq_v)
        for gi in range(CHUNK_G):
            pltpu.async_copy(table.at[gi_v.at[gi]], rows_v, sem).wait()
            pltpu.sync_copy(rows_v, acc.at[lq_v.at[gi]], add=True)

    g_base = s * GP_TILE

    @pl.loop(0, N_CHUNKS)
    def _(k):
        process(g_base + k * CHUNK_G)

    plsc.subcore_barrier()
    pltpu.sync_copy(acc.at[pl.ds(s * ZSHARE_P, ZSHARE_P)],
                    agg_out.at[c, pl.ds(s * ZSHARE_P, ZSHARE_P)])


@functools.partial(
    pl.kernel,
    out_type=(jax.ShapeDtypeStruct((4, BATCH, 128), jnp.float32),
              jax.ShapeDtypeStruct((4, BATCH, 128), jnp.float32)),
    mesh=_mesh,
    scratch_types=[
        pltpu.VMEM((128,), jnp.int32),
        pltpu.VMEM((128,), jnp.int32),
        pltpu.VMEM((128, 128), jnp.float32),
        pltpu.SemaphoreType.DMA,
    ],
)
def _sc_batch_gather(users, items, e0, e1, e2, e3, u_out, i_out,
                     uidx, iidx, grow, sem):
    c = lax.axis_index("c")
    s = lax.axis_index("s")
    w = s * NC + c
    b0 = w * (BATCH // (NC * NS))
    pltpu.sync_copy(users.at[pl.ds(b0, 128)], uidx)
    pltpu.sync_copy(items.at[pl.ds(b0, 128)], iidx)

    @pl.loop(0, 8)
    def _(j):
        iidx[pl.ds(j * 16, 16)] = iidx[pl.ds(j * 16, 16)] + N_USERS

    for l, tbl in enumerate((e0, e1, e2, e3)):
        pltpu.async_copy(tbl.at[uidx], grow, sem).wait()
        pltpu.sync_copy(grow, u_out.at[l, pl.ds(b0, 128)])
        pltpu.async_copy(tbl.at[iidx], grow, sem).wait()
        pltpu.sync_copy(grow, i_out.at[l, pl.ds(b0, 128)])


_ROWS_BLK = 1000
_N_BLKS = N_NODES // _ROWS_BLK


def _tc_prep_body(deg_ref, emb_ref, dinv_ref, emb128_ref, tab_ref):
    deg = deg_ref[...]
    dinv = jnp.where(deg > 0, lax.rsqrt(deg), 0.0)
    dinv_ref[...] = dinv
    e = emb_ref[...]
    z = jnp.zeros_like(e)
    emb128_ref[...] = jnp.concatenate([e, z], axis=1)
    es = e * dinv
    even = jnp.concatenate([es, z], axis=1)
    odd = jnp.concatenate([z, es], axis=1)
    tab_ref[...] = jnp.stack([even, odd], axis=1).reshape(2 * _ROWS_BLK, 128)


def _tc_prep(deg8, embcat):
    return pl.pallas_call(
        _tc_prep_body,
        grid=(_N_BLKS,),
        in_specs=[
            pl.BlockSpec((_ROWS_BLK, 1), lambda i: (i, 0)),
            pl.BlockSpec((_ROWS_BLK, EMB_DIM), lambda i: (i, 0)),
        ],
        out_specs=[
            pl.BlockSpec((_ROWS_BLK, 1), lambda i: (i, 0)),
            pl.BlockSpec((_ROWS_BLK, 128), lambda i: (i, 0)),
            pl.BlockSpec((2 * _ROWS_BLK, 128), lambda i: (i, 0)),
        ],
        out_shape=[
            jax.ShapeDtypeStruct((N_NODES, 1), jnp.float32),
            jax.ShapeDtypeStruct((N_NODES, 128), jnp.float32),
            jax.ShapeDtypeStruct((2 * N_NODES, 128), jnp.float32),
        ],
    )(deg8, embcat)


def _bi_interact(ragg_ref, emb_ref, dinv_ref, w1_ref, b1_ref, w2_ref, b2_ref):
    dinv = dinv_ref[...]
    agg = ragg_ref[...] * dinv
    e = emb_ref[:, :EMB_DIM]
    dn = (((1,), (1,)), ((), ()))
    y = lax.dot_general(e + agg, w1_ref[...], dn,
                        preferred_element_type=jnp.float32)
    y = y + lax.dot_general(e * agg, w2_ref[...], dn,
                            preferred_element_type=jnp.float32)
    y = y + b1_ref[...] + b2_ref[...]
    return jnp.where(y > 0, y, 0.01 * y), dinv


def _tc_transform_body(ragg_ref, emb_ref, dinv_ref, w1_ref, b1_ref,
                       w2_ref, b2_ref, out_ref, tab_ref):
    y, dinv = _bi_interact(ragg_ref, emb_ref, dinv_ref, w1_ref, b1_ref,
                           w2_ref, b2_ref)
    z = jnp.zeros_like(y)
    out_ref[...] = jnp.concatenate([y, z], axis=1)
    ys = y * dinv
    even = jnp.concatenate([ys, z], axis=1)
    odd = jnp.concatenate([z, ys], axis=1)
    tab_ref[...] = jnp.stack([even, odd], axis=1).reshape(2 * _ROWS_BLK, 128)


def _tc_transform_last_body(ragg_ref, emb_ref, dinv_ref, w1_ref, b1_ref,
                            w2_ref, b2_ref, out_ref):
    y, _ = _bi_interact(ragg_ref, emb_ref, dinv_ref, w1_ref, b1_ref,
                        w2_ref, b2_ref)
    out_ref[...] = jnp.concatenate([y, jnp.zeros_like(y)], axis=1)


_TRANSFORM_IN_SPECS = [
    pl.BlockSpec((_ROWS_BLK, EMB_DIM), lambda i: (i, 0)),
    pl.BlockSpec((_ROWS_BLK, 128), lambda i: (i, 0)),
    pl.BlockSpec((_ROWS_BLK, 1), lambda i: (i, 0)),
    pl.BlockSpec((EMB_DIM, EMB_DIM), lambda i: (0, 0)),
    pl.BlockSpec((1, EMB_DIM), lambda i: (0, 0)),
    pl.BlockSpec((EMB_DIM, EMB_DIM), lambda i: (0, 0)),
    pl.BlockSpec((1, EMB_DIM), lambda i: (0, 0)),
]


def _tc_transform(ragg, emb128, dinv, w1, b1, w2, b2):
    return pl.pallas_call(
        _tc_transform_body,
        grid=(_N_BLKS,),
        in_specs=_TRANSFORM_IN_SPECS,
        out_specs=[
            pl.BlockSpec((_ROWS_BLK, 128), lambda i: (i, 0)),
            pl.BlockSpec((2 * _ROWS_BLK, 128), lambda i: (i, 0)),
        ],
        out_shape=[
            jax.ShapeDtypeStruct((N_NODES, 128), jnp.float32),
            jax.ShapeDtypeStruct((2 * N_NODES, 128), jnp.float32),
        ],
    )(ragg, emb128, dinv, w1, b1, w2, b2)


def _tc_transform_last(ragg, emb128, dinv, w1, b1, w2, b2):
    return pl.pallas_call(
        _tc_transform_last_body,
        grid=(_N_BLKS,),
        in_specs=_TRANSFORM_IN_SPECS,
        out_specs=pl.BlockSpec((_ROWS_BLK, 128), lambda i: (i, 0)),
        out_shape=jax.ShapeDtypeStruct((N_NODES, 128), jnp.float32),
    )(ragg, emb128, dinv, w1, b1, w2, b2)


def _tc_dot_body(u_ref, i_ref, o_ref):
    p = u_ref[...] * i_ref[...]
    o_ref[...] = jnp.sum(p, axis=(0, 2))[:, None]


def _tc_dot(ug, ig):
    blk = 512
    return pl.pallas_call(
        _tc_dot_body,
        grid=(BATCH // blk,),
        in_specs=[
            pl.BlockSpec((4, blk, 128), lambda i: (0, i, 0)),
            pl.BlockSpec((4, blk, 128), lambda i: (0, i, 0)),
        ],
        out_specs=pl.BlockSpec((blk, 1), lambda i: (i, 0)),
        out_shape=jax.ShapeDtypeStruct((BATCH, 1), jnp.float32),
    )(ug, ig)


def kernel(users, items, edge_index, user_emb, item_emb, W1_w, W1_b, W2_w, W2_b):
    row = edge_index[0]
    col = edge_index[1]
    rowp = jnp.concatenate([row, jnp.full((PAD_EDGES,), N_NODES, row.dtype)])
    colp = jnp.concatenate([col, jnp.full((PAD_EDGES,), N_NODES, col.dtype)])
    base = jnp.arange(NC, dtype=col.dtype)[:, None] * HALF
    cin = jnp.logical_and(colp[None, :] >= base, colp[None, :] < base + HALF)
    lcol4 = jnp.where(cin, (colp[None, :] - base) >> 1,
                      TRASH_P).reshape(NC, GROUPS, 128)
    rin = jnp.logical_and(rowp[None, :] >= base, rowp[None, :] < base + HALF)
    lrow4 = jnp.where(rin, (rowp[None, :] - base) >> 1,
                      TRASH_P).reshape(NC, GROUPS, 128)
    dgidx2d = (rowp & 1).reshape(GROUPS, 128)
    tdeg = jnp.zeros((8, 128), jnp.float32).at[0, 0].set(1.0).at[1, 64].set(1.0)
    gidx = 2 * row + (col & 1)
    gidx2d = jnp.concatenate(
        [gidx, jnp.zeros((PAD_EDGES,), gidx.dtype)]).reshape(GROUPS, 128)
    embcat = jnp.concatenate([user_emb, item_emb], axis=0)
    zerosp = jnp.zeros((ZSHARE_P, 128), jnp.float32)

    degp = _sc_aggregate(dgidx2d, lrow4, tdeg, zerosp)
    degc = degp[:, :PAIR_HALF, :].reshape(N_NODES, EMB_DIM)[:, :1]
    dinv, emb_cur, tab = _tc_prep(degc, embcat)

    layers = [emb_cur]
    for l in range(3):
        raggp = _sc_aggregate(gidx2d, lcol4, tab, zerosp)
        ragg = raggp[:, :PAIR_HALF, :].reshape(N_NODES, EMB_DIM)
        if l < 2:
            emb_cur, tab = _tc_transform(
                ragg, emb_cur, dinv,
                W1_w[l], W1_b[l][None, :], W2_w[l], W2_b[l][None, :])
        else:
            emb_cur = _tc_transform_last(
                ragg, emb_cur, dinv,
                W1_w[l], W1_b[l][None, :], W2_w[l], W2_b[l][None, :])
        layers.append(emb_cur)

    ug, ig = _sc_batch_gather(users, items, *layers)
    scores = _tc_dot(ug, ig)
    return scores[:, 0]
